# baseline (speedup 1.0000x reference)
import functools
import math

import jax
import jax.numpy as jnp
from jax import lax
from jax.experimental import pallas as pl
from jax.experimental.pallas import tpu as pltpu
from jax.experimental.pallas import tpu_sc as plsc

D_MODEL = 64
SCALE = math.sqrt(D_MODEL)
LANES = 16
NUM_CORES = 2
NUM_SUBCORES = 16
NUM_WORKERS = NUM_CORES * NUM_SUBCORES
VOCAB = 1000000
GROUP = 128
TC_PER_TASK = 2
TASK_ROWS = GROUP * TC_PER_TASK
SUBL = 8
PAD_W = TASK_ROWS + 1
PAIR_W = 2 * D_MODEL

N_FULL_BLOCKS = VOCAB // GROUP
BASE_BLOCKS = N_FULL_BLOCKS // NUM_WORKERS
EXTRA_WORKERS = N_FULL_BLOCKS % NUM_WORKERS
TAIL_V = VOCAB - N_FULL_BLOCKS * GROUP
OUT_PAD_W = PAIR_W + 1


def _transpose_table(table4d, tail):
    mesh = plsc.VectorSubcoreMesh(core_axis_name="c", subcore_axis_name="s")

    @functools.partial(
        pl.kernel,
        out_type=jax.ShapeDtypeStruct((VOCAB // 2, PAIR_W), jnp.float32),
        mesh=mesh,
        scratch_types=[
            pltpu.VMEM((SUBL, SUBL, GROUP), jnp.float32),
            pltpu.VMEM((SUBL, SUBL, GROUP), jnp.float32),
            pltpu.VMEM((D_MODEL, OUT_PAD_W), jnp.float32),
            pltpu.VMEM((D_MODEL, OUT_PAD_W), jnp.float32),
            pltpu.SemaphoreType.DMA,
            pltpu.SemaphoreType.DMA,
            pltpu.SemaphoreType.DMA,
            pltpu.SemaphoreType.DMA,
        ],
        compiler_params=pltpu.CompilerParams(
            use_tc_tiling_on_sc=False, needs_layout_passes=False),
    )
    def tr_kernel(tbl_hbm, tail_hbm, out_hbm,
                  in_v0, in_v1, out_v0, out_v1,
                  i_sem0, i_sem1, o_sem0, o_sem1):
        in_v = (in_v0, in_v1)
        out_v = (out_v0, out_v1)
        i_sem = (i_sem0, i_sem1)
        o_sem = (o_sem0, o_sem1)

        wid = lax.axis_index("s") * NUM_CORES + lax.axis_index("c")
        base_blk = wid * BASE_BLOCKS + jnp.minimum(wid, EXTRA_WORKERS)

        iota = lax.iota(jnp.int32, LANES)
        p_vecs = [(iota >> 1) + 8 * g for g in range(GROUP // LANES)]
        c_base = (iota & 1) * D_MODEL

        def start_in(b, slot):
            pltpu.async_copy(
                tbl_hbm.at[pl.ds(0, SUBL), b], in_v[slot], i_sem[slot],
            )

        def wait_in(slot):
            pltpu.make_async_copy(
                tbl_hbm.at[pl.ds(0, SUBL), 0], in_v[slot], i_sem[slot],
            ).wait()

        def transpose_block(slot):
            blk = in_v[slot]
            outb = out_v[slot]

            @plsc.parallel_loop(0, D_MODEL, unroll=8)
            def _(k):
                c_vec = c_base + k
                tr = k // SUBL
                sb = k % SUBL
                for g in range(GROUP // LANES):
                    v = blk[tr, sb, pl.ds(g * LANES, LANES)]
                    plsc.store_scatter(outb, [p_vecs[g], c_vec], v)

        def fire_out(b, slot):
            pltpu.async_copy(
                out_v[slot].at[pl.ds(0, D_MODEL), pl.ds(0, PAIR_W)],
                out_hbm.at[pl.ds(b * D_MODEL, D_MODEL)],
                o_sem[slot],
            )

        def wait_out(slot):
            pltpu.make_async_copy(
                out_v[slot].at[pl.ds(0, D_MODEL), pl.ds(0, PAIR_W)],
                out_hbm.at[pl.ds(0, D_MODEL)],
                o_sem[slot],
            ).wait()

        def step(i, slot, first):
            wait_in(slot)
            if not first:
                wait_out(slot)
            transpose_block(slot)
            fire_out(base_blk + i, slot)

            @pl.when(i + 2 < BASE_BLOCKS)
            def _():
                start_in(base_blk + i + 2, slot)

        start_in(base_blk, 0)
        start_in(base_blk + 1, 1)
        step(0, 0, True)
        step(1, 1, True)

        @pl.loop(0, (BASE_BLOCKS - 2) // 2)
        def _(j):
            i = 2 + j * 2
            step(i, 0, False)
            step(i + 1, 1, False)

        wait_out(0)
        wait_out(1)

        @pl.when(wid < EXTRA_WORKERS)
        def _():
            b = base_blk + BASE_BLOCKS
            pltpu.sync_copy(tbl_hbm.at[pl.ds(0, SUBL), b], in_v[0])
            transpose_block(0)
            pltpu.sync_copy(
                out_v[0].at[pl.ds(0, D_MODEL), pl.ds(0, PAIR_W)],
                out_hbm.at[pl.ds(b * D_MODEL, D_MODEL)],
            )

        @pl.when(wid == NUM_WORKERS - 1)
        def _():
            dst = out_v[0].at[pl.ds(0, TAIL_V // 2), pl.ds(0, PAIR_W)]
            pltpu.sync_copy(tail_hbm, dst)
            pltpu.sync_copy(
                dst,
                out_hbm.at[pl.ds(N_FULL_BLOCKS * D_MODEL, TAIL_V // 2)],
            )

    return tr_kernel(table4d, tail)


def _gather_scale(x_t, table_lin, s, b0):
    n_tc = b0 // GROUP
    tasks_per_seq = n_tc // TC_PER_TASK
    n_tasks = s * tasks_per_seq
    tasks_per_worker = n_tasks // NUM_WORKERS
    n_tr = D_MODEL // SUBL

    mesh = plsc.VectorSubcoreMesh(core_axis_name="c", subcore_axis_name="s")

    @functools.partial(
        pl.kernel,
        out_type=jax.ShapeDtypeStruct((s, n_tr, n_tc, SUBL, GROUP),
                                      jnp.float32),
        mesh=mesh,
        scratch_types=[
            pltpu.VMEM((TASK_ROWS,), jnp.int32),
            pltpu.VMEM((TASK_ROWS,), jnp.int32),
            pltpu.VMEM((TASK_ROWS, D_MODEL), jnp.float32),
            pltpu.VMEM((TASK_ROWS, D_MODEL), jnp.float32),
            pltpu.VMEM((D_MODEL, PAD_W), jnp.float32),
            pltpu.VMEM((D_MODEL, PAD_W), jnp.float32),
            pltpu.SemaphoreType.DMA,
            pltpu.SemaphoreType.DMA,
            pltpu.SemaphoreType.DMA,
            pltpu.SemaphoreType.DMA,
            pltpu.SemaphoreType.DMA,
            pltpu.SemaphoreType.DMA,
        ],
        compiler_params=pltpu.CompilerParams(
            use_tc_tiling_on_sc=False, needs_layout_passes=False),
    )
    def emb_kernel(x_hbm, table_hbm, out_hbm,
                   idx_v0, idx_v1, rows_v0, rows_v1, out_v0, out_v1,
                   i_sem0, i_sem1, g_sem0, g_sem1, o_sem0, o_sem1):
        idx_v = (idx_v0, idx_v1)
        rows_v = (rows_v0, rows_v1)
        out_v = (out_v0, out_v1)
        i_sem = (i_sem0, i_sem1)
        g_sem = (g_sem0, g_sem1)
        o_sem = (o_sem0, o_sem1)

        wid = lax.axis_index("s") * NUM_CORES + lax.axis_index("c")
        base_task = wid * tasks_per_worker

        iota = lax.iota(jnp.int32, LANES)
        row_vecs = [iota + (m * LANES) for m in range(D_MODEL // LANES)]

        def start_idx(t, slot):
            sq = t // tasks_per_seq
            tp = t % tasks_per_seq
            pltpu.async_copy(
                x_hbm.at[sq, pl.ds(tp * TASK_ROWS, TASK_ROWS)],
                idx_v[slot], i_sem[slot],
            )

        def wait_idx(slot):
            pltpu.make_async_copy(
                x_hbm.at[0, pl.ds(0, TASK_ROWS)], idx_v[slot], i_sem[slot]
            ).wait()

        def fire_gathers(slot):
            for j in range(TC_PER_TASK):
                pltpu.async_copy(
                    table_hbm.at[idx_v[slot].at[pl.ds(j * GROUP, GROUP)]],
                    rows_v[slot].at[pl.ds(j * GROUP, GROUP)],
                    g_sem[slot],
                )

        def wait_gathers(slot):
            pltpu.make_async_copy(
                table_hbm.at[pl.ds(0, TASK_ROWS)], rows_v[slot], g_sem[slot]
            ).wait()

        def transpose_scale(slot):
            rows = rows_v[slot]
            outb = out_v[slot]

            @plsc.parallel_loop(0, TASK_ROWS, unroll=8)
            def _(j):
                col = jnp.full((LANES,), 0, jnp.int32) + j
                for m in range(D_MODEL // LANES):
                    v = rows[j, pl.ds(m * LANES, LANES)]
                    plsc.store_scatter(outb, [row_vecs[m], col], v * SCALE)

        def fire_out(t, slot):
            sq = t // tasks_per_seq
            tp = t % tasks_per_seq
            for tr in range(n_tr):
                for tcl in range(TC_PER_TASK):
                    pltpu.async_copy(
                        out_v[slot].at[pl.ds(tr * SUBL, SUBL),
                                       pl.ds(tcl * GROUP, GROUP)],
                        out_hbm.at[sq, tr, tp * TC_PER_TASK + tcl],
                        o_sem[slot],
                    )

        def wait_out(slot):
            for tr in range(n_tr):
                for tcl in range(TC_PER_TASK):
                    pltpu.make_async_copy(
                        out_v[slot].at[pl.ds(tr * SUBL, SUBL),
                                       pl.ds(tcl * GROUP, GROUP)],
                        out_hbm.at[0, tr, tcl],
                        o_sem[slot],
                    ).wait()

        pltpu.sync_copy(
            x_hbm.at[base_task // tasks_per_seq,
                     pl.ds((base_task % tasks_per_seq) * TASK_ROWS, TASK_ROWS)],
            idx_v[0],
        )
        fire_gathers(0)
        start_idx(base_task + 1, 1)
        wait_idx(1)
        fire_gathers(1)
        wait_gathers(0)
        start_idx(base_task + 2, 0)
        transpose_scale(0)
        fire_out(base_task, 0)

        def steady(t, slot):
            other = 1 - slot
            wait_out(other)
            wait_idx(other)
            fire_gathers(other)
            wait_gathers(slot)

            @pl.when(t + 2 < base_task + tasks_per_worker)
            def _():
                start_idx(t + 2, slot)

            transpose_scale(slot)
            fire_out(t, slot)

        @pl.loop(0, (tasks_per_worker - 2) // 2)
        def _(i):
            t = base_task + 1 + i * 2
            steady(t, 1)
            steady(t + 1, 0)

        t_last = base_task + tasks_per_worker - 1
        wait_out(0)
        wait_gathers(1)
        transpose_scale(1)
        fire_out(t_last, 1)
        wait_out(1)

    return emb_kernel(x_t, table_lin)


def kernel(x, table):
    b0, s = x.shape

    x_t = x.T.astype(jnp.int32)
    table4d = (
        table.T[:, : N_FULL_BLOCKS * GROUP]
        .reshape(SUBL, SUBL, N_FULL_BLOCKS, GROUP)
        .transpose(0, 2, 1, 3)
    )
    tail = table[N_FULL_BLOCKS * GROUP:, :].reshape(TAIL_V // 2, PAIR_W)

    table_pairs = _transpose_table(table4d, tail)
    table_lin = table_pairs.reshape(VOCAB, D_MODEL)
    out5d = _gather_scale(x_t, table_lin, s, b0)
    out = out5d.transpose(2, 4, 0, 1, 3).reshape(b0, s, D_MODEL)
    return out


# speedup vs baseline: 1.0438x; 1.0438x over previous
import functools
import math

import jax
import jax.numpy as jnp
from jax import lax
from jax.experimental import pallas as pl
from jax.experimental.pallas import tpu as pltpu
from jax.experimental.pallas import tpu_sc as plsc

D_MODEL = 64
SCALE = math.sqrt(D_MODEL)
LANES = 16
NUM_CORES = 2
NUM_SUBCORES = 16
NUM_WORKERS = NUM_CORES * NUM_SUBCORES
VOCAB = 1000000
GROUP = 128
TC_PER_TASK = 2
TASK_ROWS = GROUP * TC_PER_TASK
SUBL = 8
PAD_W = TASK_ROWS + 1
PAIR_W = 2 * D_MODEL

N_FULL_BLOCKS = VOCAB // GROUP
BASE_BLOCKS = N_FULL_BLOCKS // NUM_WORKERS
EXTRA_WORKERS = N_FULL_BLOCKS % NUM_WORKERS
TAIL_V = VOCAB - N_FULL_BLOCKS * GROUP
OUT_PAD_W = PAIR_W + 1


def _transpose_table(table4d, tail):
    mesh = plsc.VectorSubcoreMesh(core_axis_name="c", subcore_axis_name="s")

    @functools.partial(
        pl.kernel,
        out_type=jax.ShapeDtypeStruct((VOCAB // 2, PAIR_W), jnp.float32),
        mesh=mesh,
        scratch_types=[
            pltpu.VMEM((SUBL, SUBL, GROUP), jnp.float32),
            pltpu.VMEM((SUBL, SUBL, GROUP), jnp.float32),
            pltpu.VMEM((D_MODEL, OUT_PAD_W), jnp.float32),
            pltpu.VMEM((D_MODEL, OUT_PAD_W), jnp.float32),
            pltpu.SemaphoreType.DMA,
            pltpu.SemaphoreType.DMA,
            pltpu.SemaphoreType.DMA,
            pltpu.SemaphoreType.DMA,
        ],
        compiler_params=pltpu.CompilerParams(
            use_tc_tiling_on_sc=False, needs_layout_passes=False),
    )
    def tr_kernel(tbl_hbm, tail_hbm, out_hbm,
                  in_v0, in_v1, out_v0, out_v1,
                  i_sem0, i_sem1, o_sem0, o_sem1):
        in_v = (in_v0, in_v1)
        out_v = (out_v0, out_v1)
        i_sem = (i_sem0, i_sem1)
        o_sem = (o_sem0, o_sem1)

        wid = lax.axis_index("s") * NUM_CORES + lax.axis_index("c")
        base_blk = wid * BASE_BLOCKS + jnp.minimum(wid, EXTRA_WORKERS)

        iota = lax.iota(jnp.int32, LANES)
        p_vecs = [(iota >> 1) + 8 * g for g in range(GROUP // LANES)]
        c_base = (iota & 1) * D_MODEL

        def start_in(b, slot):
            pltpu.async_copy(
                tbl_hbm.at[pl.ds(0, SUBL), b], in_v[slot], i_sem[slot],
            )

        def wait_in(slot):
            pltpu.make_async_copy(
                tbl_hbm.at[pl.ds(0, SUBL), 0], in_v[slot], i_sem[slot],
            ).wait()

        def transpose_block(slot):
            blk = in_v[slot]
            outb = out_v[slot]

            @plsc.parallel_loop(0, D_MODEL, unroll=4)
            def _(k):
                c_vec = c_base + k
                tr = k // SUBL
                sb = k % SUBL
                for g in range(GROUP // LANES):
                    v = blk[tr, sb, pl.ds(g * LANES, LANES)]
                    plsc.store_scatter(outb, [p_vecs[g], c_vec], v)

        def fire_out(b, slot):
            pltpu.async_copy(
                out_v[slot].at[pl.ds(0, D_MODEL), pl.ds(0, PAIR_W)],
                out_hbm.at[pl.ds(b * D_MODEL, D_MODEL)],
                o_sem[slot],
            )

        def wait_out(slot):
            pltpu.make_async_copy(
                out_v[slot].at[pl.ds(0, D_MODEL), pl.ds(0, PAIR_W)],
                out_hbm.at[pl.ds(0, D_MODEL)],
                o_sem[slot],
            ).wait()

        def step(i, slot, first):
            wait_in(slot)
            if not first:
                wait_out(slot)
            transpose_block(slot)
            fire_out(base_blk + i, slot)

            @pl.when(i + 2 < BASE_BLOCKS)
            def _():
                start_in(base_blk + i + 2, slot)

        start_in(base_blk, 0)
        start_in(base_blk + 1, 1)
        step(0, 0, True)
        step(1, 1, True)

        @pl.loop(0, (BASE_BLOCKS - 2) // 2)
        def _(j):
            i = 2 + j * 2
            step(i, 0, False)
            step(i + 1, 1, False)

        wait_out(0)
        wait_out(1)

        @pl.when(wid < EXTRA_WORKERS)
        def _():
            b = base_blk + BASE_BLOCKS
            pltpu.sync_copy(tbl_hbm.at[pl.ds(0, SUBL), b], in_v[0])
            transpose_block(0)
            pltpu.sync_copy(
                out_v[0].at[pl.ds(0, D_MODEL), pl.ds(0, PAIR_W)],
                out_hbm.at[pl.ds(b * D_MODEL, D_MODEL)],
            )

        @pl.when(wid == NUM_WORKERS - 1)
        def _():
            dst = out_v[0].at[pl.ds(0, TAIL_V // 2), pl.ds(0, PAIR_W)]
            pltpu.sync_copy(tail_hbm, dst)
            pltpu.sync_copy(
                dst,
                out_hbm.at[pl.ds(N_FULL_BLOCKS * D_MODEL, TAIL_V // 2)],
            )

    return tr_kernel(table4d, tail)


def _gather_scale(x_t, table_lin, s, b0):
    n_tc = b0 // GROUP
    tasks_per_seq = n_tc // TC_PER_TASK
    n_tasks = s * tasks_per_seq
    tasks_per_worker = n_tasks // NUM_WORKERS
    n_tr = D_MODEL // SUBL

    mesh = plsc.VectorSubcoreMesh(core_axis_name="c", subcore_axis_name="s")

    @functools.partial(
        pl.kernel,
        out_type=jax.ShapeDtypeStruct((s, n_tr, n_tc, SUBL, GROUP),
                                      jnp.float32),
        mesh=mesh,
        scratch_types=[
            pltpu.VMEM((TASK_ROWS,), jnp.int32),
            pltpu.VMEM((TASK_ROWS,), jnp.int32),
            pltpu.VMEM((TASK_ROWS, D_MODEL), jnp.float32),
            pltpu.VMEM((TASK_ROWS, D_MODEL), jnp.float32),
            pltpu.VMEM((D_MODEL, PAD_W), jnp.float32),
            pltpu.VMEM((D_MODEL, PAD_W), jnp.float32),
            pltpu.SemaphoreType.DMA,
            pltpu.SemaphoreType.DMA,
            pltpu.SemaphoreType.DMA,
            pltpu.SemaphoreType.DMA,
            pltpu.SemaphoreType.DMA,
            pltpu.SemaphoreType.DMA,
        ],
        compiler_params=pltpu.CompilerParams(
            use_tc_tiling_on_sc=False, needs_layout_passes=False),
    )
    def emb_kernel(x_hbm, table_hbm, out_hbm,
                   idx_v0, idx_v1, rows_v0, rows_v1, out_v0, out_v1,
                   i_sem0, i_sem1, g_sem0, g_sem1, o_sem0, o_sem1):
        idx_v = (idx_v0, idx_v1)
        rows_v = (rows_v0, rows_v1)
        out_v = (out_v0, out_v1)
        i_sem = (i_sem0, i_sem1)
        g_sem = (g_sem0, g_sem1)
        o_sem = (o_sem0, o_sem1)

        wid = lax.axis_index("s") * NUM_CORES + lax.axis_index("c")
        base_task = wid * tasks_per_worker

        iota = lax.iota(jnp.int32, LANES)
        row_vecs = [iota + (m * LANES) for m in range(D_MODEL // LANES)]

        def start_idx(t, slot):
            sq = t // tasks_per_seq
            tp = t % tasks_per_seq
            pltpu.async_copy(
                x_hbm.at[sq, pl.ds(tp * TASK_ROWS, TASK_ROWS)],
                idx_v[slot], i_sem[slot],
            )

        def wait_idx(slot):
            pltpu.make_async_copy(
                x_hbm.at[0, pl.ds(0, TASK_ROWS)], idx_v[slot], i_sem[slot]
            ).wait()

        def fire_gathers(slot):
            for j in range(TC_PER_TASK):
                pltpu.async_copy(
                    table_hbm.at[idx_v[slot].at[pl.ds(j * GROUP, GROUP)]],
                    rows_v[slot].at[pl.ds(j * GROUP, GROUP)],
                    g_sem[slot],
                )

        def wait_gathers(slot):
            pltpu.make_async_copy(
                table_hbm.at[pl.ds(0, TASK_ROWS)], rows_v[slot], g_sem[slot]
            ).wait()

        def transpose_scale(slot):
            rows = rows_v[slot]
            outb = out_v[slot]

            @plsc.parallel_loop(0, TASK_ROWS, unroll=4)
            def _(j):
                col = jnp.full((LANES,), 0, jnp.int32) + j
                for m in range(D_MODEL // LANES):
                    v = rows[j, pl.ds(m * LANES, LANES)]
                    plsc.store_scatter(outb, [row_vecs[m], col], v * SCALE)

        def fire_out(t, slot):
            sq = t // tasks_per_seq
            tp = t % tasks_per_seq
            for tr in range(n_tr):
                for tcl in range(TC_PER_TASK):
                    pltpu.async_copy(
                        out_v[slot].at[pl.ds(tr * SUBL, SUBL),
                                       pl.ds(tcl * GROUP, GROUP)],
                        out_hbm.at[sq, tr, tp * TC_PER_TASK + tcl],
                        o_sem[slot],
                    )

        def wait_out(slot):
            for tr in range(n_tr):
                for tcl in range(TC_PER_TASK):
                    pltpu.make_async_copy(
                        out_v[slot].at[pl.ds(tr * SUBL, SUBL),
                                       pl.ds(tcl * GROUP, GROUP)],
                        out_hbm.at[0, tr, tcl],
                        o_sem[slot],
                    ).wait()

        pltpu.sync_copy(
            x_hbm.at[base_task // tasks_per_seq,
                     pl.ds((base_task % tasks_per_seq) * TASK_ROWS, TASK_ROWS)],
            idx_v[0],
        )
        fire_gathers(0)
        start_idx(base_task + 1, 1)
        wait_idx(1)
        fire_gathers(1)
        wait_gathers(0)
        start_idx(base_task + 2, 0)
        transpose_scale(0)
        fire_out(base_task, 0)

        def steady(t, slot):
            other = 1 - slot
            wait_out(other)
            wait_idx(other)
            fire_gathers(other)
            wait_gathers(slot)

            @pl.when(t + 2 < base_task + tasks_per_worker)
            def _():
                start_idx(t + 2, slot)

            transpose_scale(slot)
            fire_out(t, slot)

        @pl.loop(0, (tasks_per_worker - 2) // 2)
        def _(i):
            t = base_task + 1 + i * 2
            steady(t, 1)
            steady(t + 1, 0)

        t_last = base_task + tasks_per_worker - 1
        wait_out(0)
        wait_gathers(1)
        transpose_scale(1)
        fire_out(t_last, 1)
        wait_out(1)

    return emb_kernel(x_t, table_lin)


def kernel(x, table):
    b0, s = x.shape

    x_t = x.T.astype(jnp.int32)
    table4d = (
        table.T[:, : N_FULL_BLOCKS * GROUP]
        .reshape(SUBL, SUBL, N_FULL_BLOCKS, GROUP)
        .transpose(0, 2, 1, 3)
    )
    tail = table[N_FULL_BLOCKS * GROUP:, :].reshape(TAIL_V // 2, PAIR_W)

    table_pairs = _transpose_table(table4d, tail)
    table_lin = table_pairs.reshape(VOCAB, D_MODEL)
    out5d = _gather_scale(x_t, table_lin, s, b0)
    out = out5d.transpose(2, 4, 0, 1, 3).reshape(b0, s, D_MODEL)
    return out
